# baseline (speedup 1.0000x reference)
import functools

import jax
import jax.numpy as jnp
from jax import lax
from jax.experimental import pallas as pl
from jax.experimental.pallas import tpu as pltpu
from jax.experimental.pallas import tpu_sc as plsc

_D = 768
_BATCH = 4
_SEQ = 2048
_NC = 2
_NS = 16
_NW = _NC * _NS
_PP = _SEQ // _NW
_C = 8
_NSTEP = _PP // _C
_ROWS = _BATCH * _C
_LPT = _D // 16


@functools.partial(
    pl.kernel,
    out_type=jax.ShapeDtypeStruct((_BATCH, _SEQ, _D), jnp.float32),
    mesh=plsc.VectorSubcoreMesh(core_axis_name="c", subcore_axis_name="s"),
    scratch_types=[
        pltpu.VMEM((_NSTEP, _ROWS), jnp.int32),
        [pltpu.VMEM((_ROWS, _D), jnp.float32) for _ in range(3)],
        pltpu.VMEM((_PP, _D), jnp.float32),
        [pltpu.SemaphoreType.DMA for _ in range(3)],
        [pltpu.SemaphoreType.DMA for _ in range(3)],
        pltpu.SemaphoreType.DMA,
        pltpu.SemaphoreType.DMA,
    ],
)
def _emb_kernel(x_hbm, wte_hbm, wpe_hbm, out_hbm,
                idx2_v, rows, wpe_v, gsem, ssem, isem, wsem):
    wid = lax.axis_index("s") * _NC + lax.axis_index("c")
    pos0 = wid * _PP

    idx_cp = pltpu.async_copy(x_hbm.at[wid], idx2_v, isem)
    wpe_cp = pltpu.async_copy(wpe_hbm.at[pl.ds(pos0, _PP)], wpe_v, wsem)
    idx_cp.wait()

    def gather(i):
        return pltpu.async_copy(wte_hbm.at[idx2_v.at[i]], rows[i % 3],
                                gsem[i % 3])

    g_cp = {0: gather(0), 1: gather(1)}
    s_cp = {}
    for i in range(_NSTEP):
        if i == 0:
            wpe_cp.wait()
        g_cp[i].wait()
        buf = rows[i % 3]

        def tok_body(t, carry):
            for dd in range(_LPT):
                sl = pl.ds(dd * 16, 16)
                wv = wpe_v[i * _C + t, sl]
                for b in range(_BATCH):
                    r = b * _C + t
                    buf[r, sl] = buf[r, sl] + wv
            return carry

        lax.fori_loop(0, _C, tok_body, 0)
        if i + 2 < _NSTEP:
            if i - 1 >= 0:
                for cp in s_cp[i - 1]:
                    cp.wait()
            g_cp[i + 2] = gather(i + 2)
        s_cp[i] = [
            pltpu.async_copy(
                buf.at[pl.ds(b * _C, _C)],
                out_hbm.at[b, pl.ds(pos0 + i * _C, _C)], ssem[i % 3])
            for b in range(_BATCH)
        ]
    for i in range(_NSTEP - 3, _NSTEP):
        for cp in s_cp[i]:
            cp.wait()


def kernel(X, wte, wpe):
    xs = (X.astype(jnp.int32)
          .reshape(_BATCH, _NW, _NSTEP, _C)
          .transpose(1, 2, 0, 3)
          .reshape(_NW, _NSTEP, _ROWS))
    return _emb_kernel(xs, wte, wpe)
